# baseline (device time: 75953 ns/iter reference)
import jax
import jax.numpy as jnp
from jax import lax
from jax.experimental import pallas as pl
from jax.experimental.pallas import tpu as pltpu

N_DEV = 16


def kernel(x, w_mat):
    m, k_per = x.shape
    _, n = w_mat.shape
    m_blk = m // N_DEV

    def body(x_ref, w_ref, out_ref, p_ref, send_ref, recv_ref, send_sem, recv_sems):
        my = lax.axis_index("i")
        left = (my - 1) % N_DEV
        right = (my + 1) % N_DEV

        barrier_sem = pltpu.get_barrier_semaphore()
        for nbr in (left, right):
            pl.semaphore_signal(
                barrier_sem, inc=1,
                device_id=(nbr,), device_id_type=pl.DeviceIdType.MESH,
            )
        pl.semaphore_wait(barrier_sem, 2)

        p_ref[...] = jnp.dot(
            x_ref[...], w_ref[...], preferred_element_type=jnp.float32
        )

        for s in range(N_DEV - 1):
            b = (my - s - 1) % N_DEV
            chunk = p_ref[pl.ds(b * m_blk, m_blk), :]
            if s == 0:
                send_ref[...] = chunk
            else:
                send_ref[...] = chunk + recv_ref[s - 1]
            rdma = pltpu.make_async_remote_copy(
                src_ref=send_ref,
                dst_ref=recv_ref.at[s],
                send_sem=send_sem,
                recv_sem=recv_sems.at[s],
                device_id=(right,),
                device_id_type=pl.DeviceIdType.MESH,
            )
            rdma.start()
            rdma.wait()

        y = p_ref[pl.ds(my * m_blk, m_blk), :] + recv_ref[N_DEV - 2]
        out_ref[...] = y * (1.0 / (1.0 + jnp.exp(-y)))

    out_shape = jax.ShapeDtypeStruct((m_blk, n), jnp.float32)
    return pl.pallas_call(
        body,
        out_shape=out_shape,
        in_specs=[
            pl.BlockSpec(memory_space=pltpu.VMEM),
            pl.BlockSpec(memory_space=pltpu.VMEM),
        ],
        out_specs=pl.BlockSpec(memory_space=pltpu.VMEM),
        scratch_shapes=[
            pltpu.VMEM((m, n), jnp.float32),
            pltpu.VMEM((m_blk, n), jnp.float32),
            pltpu.VMEM((N_DEV - 1, m_blk, n), jnp.float32),
            pltpu.SemaphoreType.DMA,
            pltpu.SemaphoreType.DMA((N_DEV - 1,)),
        ],
        compiler_params=pltpu.CompilerParams(collective_id=0),
    )(x, w_mat)


# device time: 43342 ns/iter; 1.7524x vs baseline; 1.7524x over previous
import jax
import jax.numpy as jnp
from jax import lax
from jax.experimental import pallas as pl
from jax.experimental.pallas import tpu as pltpu

N_DEV = 16
PLANE = 4
ZDIM = 4


def kernel(x, w_mat):
    m, k_per = x.shape
    _, n = w_mat.shape
    m_blk = m // N_DEV
    n2 = n // 2

    def body(x_ref, w_ref, out_ref, p_ref,
             s1r, s1l, r1r, r1l, s2u, s2d, r2u, r2d,
             sem_s1r, sem_s1l, sem_r1r, sem_r1l,
             sem_s2u, sem_s2d, sem_r2u, sem_r2d):
        my = lax.axis_index("i")
        z = my // PLANE
        p = my % PLANE
        plane_r = z * PLANE + (p + 1) % PLANE
        plane_l = z * PLANE + (p - 1) % PLANE
        z_up = ((z + 1) % ZDIM) * PLANE + p
        z_dn = ((z - 1) % ZDIM) * PLANE + p

        barrier_sem = pltpu.get_barrier_semaphore()
        for nbr in (plane_r, plane_l, z_up, z_dn):
            pl.semaphore_signal(
                barrier_sem, inc=1,
                device_id=(nbr,), device_id_type=pl.DeviceIdType.MESH,
            )
        pl.semaphore_wait(barrier_sem, 4)

        p_ref[...] = jnp.dot(
            x_ref[...], w_ref[...], preferred_element_type=jnp.float32
        )

        cols_r = pl.ds(0, n2)
        cols_l = pl.ds(n2, n2)

        for s in range(PLANE - 1):
            qr = (p - s - 1) % PLANE
            ql = (p + s + 1) % PLANE
            for zz in range(ZDIM):
                rows_r = pl.ds((zz * PLANE + qr) * m_blk, m_blk)
                rows_l = pl.ds((zz * PLANE + ql) * m_blk, m_blk)
                if s == 0:
                    s1r[zz, :, :] = p_ref[rows_r, cols_r]
                    s1l[zz, :, :] = p_ref[rows_l, cols_l]
                else:
                    s1r[zz, :, :] = p_ref[rows_r, cols_r] + r1r[s - 1, zz, :, :]
                    s1l[zz, :, :] = p_ref[rows_l, cols_l] + r1l[s - 1, zz, :, :]
            rdma_r = pltpu.make_async_remote_copy(
                src_ref=s1r, dst_ref=r1r.at[s],
                send_sem=sem_s1r, recv_sem=sem_r1r.at[s],
                device_id=(plane_r,), device_id_type=pl.DeviceIdType.MESH,
            )
            rdma_l = pltpu.make_async_remote_copy(
                src_ref=s1l, dst_ref=r1l.at[s],
                send_sem=sem_s1l, recv_sem=sem_r1l.at[s],
                device_id=(plane_l,), device_id_type=pl.DeviceIdType.MESH,
            )
            rdma_r.start()
            rdma_l.start()
            rdma_r.wait()
            rdma_l.wait()


        for t in range(ZDIM - 1):
            ju = (z - t - 1) % ZDIM
            jd = (z + t + 1) % ZDIM
            rows_u = pl.ds((ju * PLANE + p) * m_blk, m_blk)
            rows_d = pl.ds((jd * PLANE + p) * m_blk, m_blk)
            base_u = p_ref[rows_u, cols_r] + r1r[PLANE - 2, ju, :, :]
            base_d = p_ref[rows_d, cols_l] + r1l[PLANE - 2, jd, :, :]
            if t == 0:
                s2u[...] = base_u
                s2d[...] = base_d
            else:
                s2u[...] = base_u + r2u[t - 1, :, :]
                s2d[...] = base_d + r2d[t - 1, :, :]
            rdma_u = pltpu.make_async_remote_copy(
                src_ref=s2u, dst_ref=r2u.at[t],
                send_sem=sem_s2u, recv_sem=sem_r2u.at[t],
                device_id=(z_up,), device_id_type=pl.DeviceIdType.MESH,
            )
            rdma_d = pltpu.make_async_remote_copy(
                src_ref=s2d, dst_ref=r2d.at[t],
                send_sem=sem_s2d, recv_sem=sem_r2d.at[t],
                device_id=(z_dn,), device_id_type=pl.DeviceIdType.MESH,
            )
            rdma_u.start()
            rdma_d.start()
            rdma_u.wait()
            rdma_d.wait()

        rows_m = pl.ds(my * m_blk, m_blk)
        yu = p_ref[rows_m, cols_r] + r1r[PLANE - 2, z, :, :] + r2u[ZDIM - 2, :, :]
        yd = p_ref[rows_m, cols_l] + r1l[PLANE - 2, z, :, :] + r2d[ZDIM - 2, :, :]
        out_ref[:, cols_r] = yu * (1.0 / (1.0 + jnp.exp(-yu)))
        out_ref[:, cols_l] = yd * (1.0 / (1.0 + jnp.exp(-yd)))

    out_shape = jax.ShapeDtypeStruct((m_blk, n), jnp.float32)
    return pl.pallas_call(
        body,
        out_shape=out_shape,
        in_specs=[
            pl.BlockSpec(memory_space=pltpu.VMEM),
            pl.BlockSpec(memory_space=pltpu.VMEM),
        ],
        out_specs=pl.BlockSpec(memory_space=pltpu.VMEM),
        scratch_shapes=[
            pltpu.VMEM((m, n), jnp.float32),
            pltpu.VMEM((ZDIM, m_blk, n2), jnp.float32),
            pltpu.VMEM((ZDIM, m_blk, n2), jnp.float32),
            pltpu.VMEM((PLANE - 1, ZDIM, m_blk, n2), jnp.float32),
            pltpu.VMEM((PLANE - 1, ZDIM, m_blk, n2), jnp.float32),
            pltpu.VMEM((m_blk, n2), jnp.float32),
            pltpu.VMEM((m_blk, n2), jnp.float32),
            pltpu.VMEM((ZDIM - 1, m_blk, n2), jnp.float32),
            pltpu.VMEM((ZDIM - 1, m_blk, n2), jnp.float32),
            pltpu.SemaphoreType.DMA,
            pltpu.SemaphoreType.DMA,
            pltpu.SemaphoreType.DMA((PLANE - 1,)),
            pltpu.SemaphoreType.DMA((PLANE - 1,)),
            pltpu.SemaphoreType.DMA,
            pltpu.SemaphoreType.DMA,
            pltpu.SemaphoreType.DMA((ZDIM - 1,)),
            pltpu.SemaphoreType.DMA((ZDIM - 1,)),
        ],
        compiler_params=pltpu.CompilerParams(collective_id=0),
    )(x, w_mat)


# device time: 35902 ns/iter; 2.1156x vs baseline; 1.2072x over previous
import jax
import jax.numpy as jnp
from jax import lax
from jax.experimental import pallas as pl
from jax.experimental.pallas import tpu as pltpu

N_DEV = 16
PLANE = 4
ZDIM = 4


def kernel(x, w_mat):
    m, k_per = x.shape
    _, n = w_mat.shape
    m_blk = m // N_DEV
    n2 = n // 2

    def body(x_ref, w_ref, out_ref, p_ref,
             s1r, s1l, r1r, r1l, s2u, s2d, r2u, r2d,
             sem_s1r, sem_s1l, sem_r1r, sem_r1l,
             sem_s2u, sem_s2d, sem_r2u, sem_r2d):
        my = lax.axis_index("i")
        z = my // PLANE
        p = my % PLANE
        plane_r = z * PLANE + (p + 1) % PLANE
        plane_l = z * PLANE + (p - 1) % PLANE
        z_up = ((z + 1) % ZDIM) * PLANE + p
        z_dn = ((z - 1) % ZDIM) * PLANE + p

        cols_r = pl.ds(0, n2)
        cols_l = pl.ds(n2, n2)

        def p1(bufs, s, zz, dev):
            send, recv, ssem, rsem = bufs
            return pltpu.make_async_remote_copy(
                src_ref=send.at[s, zz], dst_ref=recv.at[s, zz],
                send_sem=ssem.at[s, zz], recv_sem=rsem.at[s, zz],
                device_id=(dev,), device_id_type=pl.DeviceIdType.MESH,
            )

        def p2(send, recv, ssem, rsem, t, dev):
            return pltpu.make_async_remote_copy(
                src_ref=send.at[t], dst_ref=recv.at[t],
                send_sem=ssem.at[t], recv_sem=rsem.at[t],
                device_id=(dev,), device_id_type=pl.DeviceIdType.MESH,
            )

        bufs_r = (s1r, r1r, sem_s1r, sem_r1r)
        bufs_l = (s1l, r1l, sem_s1l, sem_r1l)

        barrier_sem = pltpu.get_barrier_semaphore()
        for nbr in (plane_r, plane_l, z_up, z_dn):
            pl.semaphore_signal(
                barrier_sem, inc=1,
                device_id=(nbr,), device_id_type=pl.DeviceIdType.MESH,
            )
        pl.semaphore_wait(barrier_sem, 4)

        p_ref[...] = jnp.dot(
            x_ref[...], w_ref[...], preferred_element_type=jnp.float32
        )

        for s in range(PLANE - 1):
            qr = (p - s - 1) % PLANE
            ql = (p + s + 1) % PLANE
            for k in range(ZDIM):
                zr = (z - 1 - k) % ZDIM
                zl = (z + 1 + k) % ZDIM
                rows_r = pl.ds((zr * PLANE + qr) * m_blk, m_blk)
                rows_l = pl.ds((zl * PLANE + ql) * m_blk, m_blk)
                if s == 0:
                    s1r[s, zr, :, :] = p_ref[rows_r, cols_r]
                    s1l[s, zl, :, :] = p_ref[rows_l, cols_l]
                else:
                    p1(bufs_r, s - 1, zr, plane_r).wait_recv()
                    p1(bufs_l, s - 1, zl, plane_l).wait_recv()
                    s1r[s, zr, :, :] = p_ref[rows_r, cols_r] + r1r[s - 1, zr, :, :]
                    s1l[s, zl, :, :] = p_ref[rows_l, cols_l] + r1l[s - 1, zl, :, :]
                p1(bufs_r, s, zr, plane_r).start()
                p1(bufs_l, s, zl, plane_l).start()

        last = PLANE - 2
        for t in range(ZDIM - 1):
            ju = (z - 1 - t) % ZDIM
            jd = (z + 1 + t) % ZDIM
            p1(bufs_r, last, ju, plane_r).wait_recv()
            p1(bufs_l, last, jd, plane_l).wait_recv()
            rows_u = pl.ds((ju * PLANE + p) * m_blk, m_blk)
            rows_d = pl.ds((jd * PLANE + p) * m_blk, m_blk)
            base_u = p_ref[rows_u, cols_r] + r1r[last, ju, :, :]
            base_d = p_ref[rows_d, cols_l] + r1l[last, jd, :, :]
            if t > 0:
                p2(s2u, r2u, sem_s2u, sem_r2u, t - 1, z_up).wait_recv()
                p2(s2d, r2d, sem_s2d, sem_r2d, t - 1, z_dn).wait_recv()
                base_u = base_u + r2u[t - 1, :, :]
                base_d = base_d + r2d[t - 1, :, :]
            s2u[t, :, :] = base_u
            s2d[t, :, :] = base_d
            p2(s2u, r2u, sem_s2u, sem_r2u, t, z_up).start()
            p2(s2d, r2d, sem_s2d, sem_r2d, t, z_dn).start()

        p1(bufs_r, last, z, plane_r).wait_recv()
        p1(bufs_l, last, z, plane_l).wait_recv()
        p2(s2u, r2u, sem_s2u, sem_r2u, ZDIM - 2, z_up).wait_recv()
        p2(s2d, r2d, sem_s2d, sem_r2d, ZDIM - 2, z_dn).wait_recv()
        rows_m = pl.ds(my * m_blk, m_blk)
        yu = p_ref[rows_m, cols_r] + r1r[last, z, :, :] + r2u[ZDIM - 2, :, :]
        yd = p_ref[rows_m, cols_l] + r1l[last, z, :, :] + r2d[ZDIM - 2, :, :]
        out_ref[:, cols_r] = yu * (1.0 / (1.0 + jnp.exp(-yu)))
        out_ref[:, cols_l] = yd * (1.0 / (1.0 + jnp.exp(-yd)))

        for s in range(PLANE - 1):
            for zz in range(ZDIM):
                p1(bufs_r, s, zz, plane_r).wait_send()
                p1(bufs_l, s, zz, plane_l).wait_send()
        for t in range(ZDIM - 1):
            p2(s2u, r2u, sem_s2u, sem_r2u, t, z_up).wait_send()
            p2(s2d, r2d, sem_s2d, sem_r2d, t, z_dn).wait_send()

    out_shape = jax.ShapeDtypeStruct((m_blk, n), jnp.float32)
    return pl.pallas_call(
        body,
        out_shape=out_shape,
        in_specs=[
            pl.BlockSpec(memory_space=pltpu.VMEM),
            pl.BlockSpec(memory_space=pltpu.VMEM),
        ],
        out_specs=pl.BlockSpec(memory_space=pltpu.VMEM),
        scratch_shapes=[
            pltpu.VMEM((m, n), jnp.float32),
            pltpu.VMEM((PLANE - 1, ZDIM, m_blk, n2), jnp.float32),
            pltpu.VMEM((PLANE - 1, ZDIM, m_blk, n2), jnp.float32),
            pltpu.VMEM((PLANE - 1, ZDIM, m_blk, n2), jnp.float32),
            pltpu.VMEM((PLANE - 1, ZDIM, m_blk, n2), jnp.float32),
            pltpu.VMEM((ZDIM - 1, m_blk, n2), jnp.float32),
            pltpu.VMEM((ZDIM - 1, m_blk, n2), jnp.float32),
            pltpu.VMEM((ZDIM - 1, m_blk, n2), jnp.float32),
            pltpu.VMEM((ZDIM - 1, m_blk, n2), jnp.float32),
            pltpu.SemaphoreType.DMA((PLANE - 1, ZDIM)),
            pltpu.SemaphoreType.DMA((PLANE - 1, ZDIM)),
            pltpu.SemaphoreType.DMA((PLANE - 1, ZDIM)),
            pltpu.SemaphoreType.DMA((PLANE - 1, ZDIM)),
            pltpu.SemaphoreType.DMA((ZDIM - 1,)),
            pltpu.SemaphoreType.DMA((ZDIM - 1,)),
            pltpu.SemaphoreType.DMA((ZDIM - 1,)),
            pltpu.SemaphoreType.DMA((ZDIM - 1,)),
        ],
        compiler_params=pltpu.CompilerParams(collective_id=0),
    )(x, w_mat)


# device time: 33293 ns/iter; 2.2814x vs baseline; 1.0784x over previous
import jax
import jax.numpy as jnp
from jax import lax
from jax.experimental import pallas as pl
from jax.experimental.pallas import tpu as pltpu

N_DEV = 16
PLANE = 4
ZDIM = 4
NA = 768
NB = 256


def kernel(x, w_mat):
    m, k_per = x.shape
    _, n = w_mat.shape
    m_blk = m // N_DEV
    na2 = NA // 2
    nb2 = NB // 2
    sup = m // ZDIM

    def body(x_ref, w_ref, out_ref, p_ref,
             s1r, s1l, r1r, r1l, s2u, s2d, r2u, r2d,
             b1us, b1ds, b1ur, b1dr, b2rs, b2ls, b2rr, b2lr,
             sem_s1r, sem_s1l, sem_r1r, sem_r1l,
             sem_s2u, sem_s2d, sem_r2u, sem_r2d,
             sem_b1us, sem_b1ds, sem_b1ur, sem_b1dr,
             sem_b2rs, sem_b2ls, sem_b2rr, sem_b2lr):
        my = lax.axis_index("i")
        z = my // PLANE
        p = my % PLANE
        plane_r = z * PLANE + (p + 1) % PLANE
        plane_l = z * PLANE + (p - 1) % PLANE
        z_up = ((z + 1) % ZDIM) * PLANE + p
        z_dn = ((z - 1) % ZDIM) * PLANE + p

        cols_a_r = pl.ds(0, na2)
        cols_a_l = pl.ds(na2, na2)
        cols_b_u = pl.ds(NA, nb2)
        cols_b_d = pl.ds(NA + nb2, nb2)

        def rc(send, recv, ssem, rsem, idx, dev):
            return pltpu.make_async_remote_copy(
                src_ref=send.at[idx], dst_ref=recv.at[idx],
                send_sem=ssem.at[idx], recv_sem=rsem.at[idx],
                device_id=(dev,), device_id_type=pl.DeviceIdType.MESH,
            )

        def a1(bufs, s, zz, dev):
            send, recv, ssem, rsem = bufs
            return pltpu.make_async_remote_copy(
                src_ref=send.at[s, zz], dst_ref=recv.at[s, zz],
                send_sem=ssem.at[s, zz], recv_sem=rsem.at[s, zz],
                device_id=(dev,), device_id_type=pl.DeviceIdType.MESH,
            )

        bufs_ar = (s1r, r1r, sem_s1r, sem_r1r)
        bufs_al = (s1l, r1l, sem_s1l, sem_r1l)

        barrier_sem = pltpu.get_barrier_semaphore()
        for nbr in (plane_r, plane_l, z_up, z_dn):
            pl.semaphore_signal(
                barrier_sem, inc=1,
                device_id=(nbr,), device_id_type=pl.DeviceIdType.MESH,
            )
        pl.semaphore_wait(barrier_sem, 4)

        p_ref[...] = jnp.dot(
            x_ref[...], w_ref[...], preferred_element_type=jnp.float32
        )

        for s in range(3):
            ju_b = (z - s - 1) % ZDIM
            jd_b = (z + s + 1) % ZDIM
            rows_bu = pl.ds(ju_b * sup, sup)
            rows_bd = pl.ds(jd_b * sup, sup)
            if s == 0:
                b1us[s, :, :] = p_ref[rows_bu, cols_b_u]
                b1ds[s, :, :] = p_ref[rows_bd, cols_b_d]
            else:
                rc(b1us, b1ur, sem_b1us, sem_b1ur, s - 1, z_up).wait_recv()
                rc(b1ds, b1dr, sem_b1ds, sem_b1dr, s - 1, z_dn).wait_recv()
                b1us[s, :, :] = p_ref[rows_bu, cols_b_u] + b1ur[s - 1, :, :]
                b1ds[s, :, :] = p_ref[rows_bd, cols_b_d] + b1dr[s - 1, :, :]
            rc(b1us, b1ur, sem_b1us, sem_b1ur, s, z_up).start()
            rc(b1ds, b1dr, sem_b1ds, sem_b1dr, s, z_dn).start()

            qr = (p - s - 1) % PLANE
            ql = (p + s + 1) % PLANE
            for k in range(ZDIM):
                zr = (z - 1 - k) % ZDIM
                zl = (z + 1 + k) % ZDIM
                rows_r = pl.ds((zr * PLANE + qr) * m_blk, m_blk)
                rows_l = pl.ds((zl * PLANE + ql) * m_blk, m_blk)
                if s == 0:
                    s1r[s, zr, :, :] = p_ref[rows_r, cols_a_r]
                    s1l[s, zl, :, :] = p_ref[rows_l, cols_a_l]
                else:
                    a1(bufs_ar, s - 1, zr, plane_r).wait_recv()
                    a1(bufs_al, s - 1, zl, plane_l).wait_recv()
                    s1r[s, zr, :, :] = p_ref[rows_r, cols_a_r] + r1r[s - 1, zr, :, :]
                    s1l[s, zl, :, :] = p_ref[rows_l, cols_a_l] + r1l[s - 1, zl, :, :]
                a1(bufs_ar, s, zr, plane_r).start()
                a1(bufs_al, s, zl, plane_l).start()

        for t in range(3):
            ju = (z - 1 - t) % ZDIM
            jd = (z + 1 + t) % ZDIM
            a1(bufs_ar, 2, ju, plane_r).wait_recv()
            a1(bufs_al, 2, jd, plane_l).wait_recv()
            rows_u = pl.ds((ju * PLANE + p) * m_blk, m_blk)
            rows_d = pl.ds((jd * PLANE + p) * m_blk, m_blk)
            base_u = p_ref[rows_u, cols_a_r] + r1r[2, ju, :, :]
            base_d = p_ref[rows_d, cols_a_l] + r1l[2, jd, :, :]
            if t > 0:
                rc(s2u, r2u, sem_s2u, sem_r2u, t - 1, z_up).wait_recv()
                rc(s2d, r2d, sem_s2d, sem_r2d, t - 1, z_dn).wait_recv()
                base_u = base_u + r2u[t - 1, :, :]
                base_d = base_d + r2d[t - 1, :, :]
            s2u[t, :, :] = base_u
            s2d[t, :, :] = base_d
            rc(s2u, r2u, sem_s2u, sem_r2u, t, z_up).start()
            rc(s2d, r2d, sem_s2d, sem_r2d, t, z_dn).start()

            if t == 0:
                rc(b1us, b1ur, sem_b1us, sem_b1ur, 2, z_up).wait_recv()
                rc(b1ds, b1dr, sem_b1ds, sem_b1dr, 2, z_dn).wait_recv()
            qbr = (p - t - 1) % PLANE
            qbl = (p + t + 1) % PLANE
            rows_br = pl.ds((z * PLANE + qbr) * m_blk, m_blk)
            rows_bl = pl.ds((z * PLANE + qbl) * m_blk, m_blk)
            base_br = p_ref[rows_br, cols_b_u] + b1ur[2, pl.ds(qbr * m_blk, m_blk), :]
            base_bl = p_ref[rows_bl, cols_b_d] + b1dr[2, pl.ds(qbl * m_blk, m_blk), :]
            if t > 0:
                rc(b2rs, b2rr, sem_b2rs, sem_b2rr, t - 1, plane_r).wait_recv()
                rc(b2ls, b2lr, sem_b2ls, sem_b2lr, t - 1, plane_l).wait_recv()
                base_br = base_br + b2rr[t - 1, :, :]
                base_bl = base_bl + b2lr[t - 1, :, :]
            b2rs[t, :, :] = base_br
            b2ls[t, :, :] = base_bl
            rc(b2rs, b2rr, sem_b2rs, sem_b2rr, t, plane_r).start()
            rc(b2ls, b2lr, sem_b2ls, sem_b2lr, t, plane_l).start()

        a1(bufs_ar, 2, z, plane_r).wait_recv()
        a1(bufs_al, 2, z, plane_l).wait_recv()
        rc(s2u, r2u, sem_s2u, sem_r2u, 2, z_up).wait_recv()
        rc(s2d, r2d, sem_s2d, sem_r2d, 2, z_dn).wait_recv()
        rc(b2rs, b2rr, sem_b2rs, sem_b2rr, 2, plane_r).wait_recv()
        rc(b2ls, b2lr, sem_b2ls, sem_b2lr, 2, plane_l).wait_recv()
        rows_m = pl.ds(my * m_blk, m_blk)
        rows_in_sup = pl.ds(p * m_blk, m_blk)
        yar = p_ref[rows_m, cols_a_r] + r1r[2, z, :, :] + r2u[2, :, :]
        yal = p_ref[rows_m, cols_a_l] + r1l[2, z, :, :] + r2d[2, :, :]
        ybr = p_ref[rows_m, cols_b_u] + b1ur[2, rows_in_sup, :] + b2rr[2, :, :]
        ybl = p_ref[rows_m, cols_b_d] + b1dr[2, rows_in_sup, :] + b2lr[2, :, :]
        out_ref[:, cols_a_r] = yar * (1.0 / (1.0 + jnp.exp(-yar)))
        out_ref[:, cols_a_l] = yal * (1.0 / (1.0 + jnp.exp(-yal)))
        out_ref[:, cols_b_u] = ybr * (1.0 / (1.0 + jnp.exp(-ybr)))
        out_ref[:, cols_b_d] = ybl * (1.0 / (1.0 + jnp.exp(-ybl)))

        for s in range(3):
            for zz in range(ZDIM):
                a1(bufs_ar, s, zz, plane_r).wait_send()
                a1(bufs_al, s, zz, plane_l).wait_send()
            rc(b1us, b1ur, sem_b1us, sem_b1ur, s, z_up).wait_send()
            rc(b1ds, b1dr, sem_b1ds, sem_b1dr, s, z_dn).wait_send()
            rc(s2u, r2u, sem_s2u, sem_r2u, s, z_up).wait_send()
            rc(s2d, r2d, sem_s2d, sem_r2d, s, z_dn).wait_send()
            rc(b2rs, b2rr, sem_b2rs, sem_b2rr, s, plane_r).wait_send()
            rc(b2ls, b2lr, sem_b2ls, sem_b2lr, s, plane_l).wait_send()

    out_shape = jax.ShapeDtypeStruct((m_blk, n), jnp.float32)
    dma = pltpu.SemaphoreType.DMA
    return pl.pallas_call(
        body,
        out_shape=out_shape,
        in_specs=[
            pl.BlockSpec(memory_space=pltpu.VMEM),
            pl.BlockSpec(memory_space=pltpu.VMEM),
        ],
        out_specs=pl.BlockSpec(memory_space=pltpu.VMEM),
        scratch_shapes=[
            pltpu.VMEM((m, n), jnp.float32),
            pltpu.VMEM((3, ZDIM, m_blk, na2), jnp.float32),
            pltpu.VMEM((3, ZDIM, m_blk, na2), jnp.float32),
            pltpu.VMEM((3, ZDIM, m_blk, na2), jnp.float32),
            pltpu.VMEM((3, ZDIM, m_blk, na2), jnp.float32),
            pltpu.VMEM((3, m_blk, na2), jnp.float32),
            pltpu.VMEM((3, m_blk, na2), jnp.float32),
            pltpu.VMEM((3, m_blk, na2), jnp.float32),
            pltpu.VMEM((3, m_blk, na2), jnp.float32),
            pltpu.VMEM((3, sup, nb2), jnp.float32),
            pltpu.VMEM((3, sup, nb2), jnp.float32),
            pltpu.VMEM((3, sup, nb2), jnp.float32),
            pltpu.VMEM((3, sup, nb2), jnp.float32),
            pltpu.VMEM((3, m_blk, nb2), jnp.float32),
            pltpu.VMEM((3, m_blk, nb2), jnp.float32),
            pltpu.VMEM((3, m_blk, nb2), jnp.float32),
            pltpu.VMEM((3, m_blk, nb2), jnp.float32),
            dma((3, ZDIM)), dma((3, ZDIM)), dma((3, ZDIM)), dma((3, ZDIM)),
            dma((3,)), dma((3,)), dma((3,)), dma((3,)),
            dma((3,)), dma((3,)), dma((3,)), dma((3,)),
            dma((3,)), dma((3,)), dma((3,)), dma((3,)),
        ],
        compiler_params=pltpu.CompilerParams(collective_id=0),
    )(x, w_mat)
